# baseline (speedup 1.0000x reference)
import functools

import jax
import jax.numpy as jnp
from jax import lax
from jax.experimental import pallas as pl
from jax.experimental.pallas import tpu as pltpu
from jax.experimental.pallas import tpu_sc as plsc

B, L, V, D = 1024, 200, 1000000, 64
NW = 32
ROWS_W = B // NW
RC = 4
NCH = ROWS_W // RC
HALVES = ((0, 104), (104, 96))
LANES = 16
XB = 512
NXB = (V + XB - 1) // XB
V2 = (XB // 2) * NXB


def _comb_body(seg_ref, pos_ref, out_ref):
    out_ref[...] = seg_ref[...][:, None, :] + pos_ref[...][None, :, :]


def _build_comb(seg_table, pos200):
    return pl.pallas_call(
        _comb_body,
        out_shape=jax.ShapeDtypeStruct((3, L, D), jnp.float32),
    )(seg_table, pos200)


def _relayout_body(tt_ref, out_ref):
    a = jnp.swapaxes(tt_ref[...], 0, 1)
    out_ref[:, :D] = a[: XB // 2, :]
    out_ref[:, D:] = a[XB // 2:, :]


def _relayout(tt):
    return pl.pallas_call(
        _relayout_body,
        grid=(NXB,),
        in_specs=[pl.BlockSpec((D, XB), lambda i: (0, i))],
        out_specs=pl.BlockSpec((XB // 2, 2 * D), lambda i: (i, 0)),
        out_shape=jax.ShapeDtypeStruct((V2, 2 * D), jnp.float32),
    )(tt)


_MESH = plsc.VectorSubcoreMesh(core_axis_name="c", subcore_axis_name="s")


@functools.partial(
    pl.kernel,
    mesh=_MESH,
    compiler_params=pltpu.CompilerParams(use_tc_tiling_on_sc=False),
    out_type=jax.ShapeDtypeStruct((B, L, D), jnp.float32),
    scratch_types=[
        pltpu.VMEM((RC, L), jnp.int32),
        pltpu.VMEM((RC, L), jnp.int32),
        pltpu.VMEM((RC, L), jnp.int32),
        pltpu.VMEM((RC, L), jnp.int32),
        pltpu.VMEM((RC, L, D), jnp.float32),
        pltpu.SemaphoreType.DMA,
        pltpu.SemaphoreType.DMA,
    ],
)
def _sc_embed(x_hbm, sg_hbm, table_hbm, comb_hbm, out_hbm,
              idx_v, tidx_v, sidx_v, cidx_v, rows_v, sem, sem2):
    cid = lax.axis_index("c")
    sid = lax.axis_index("s")
    wid = sid * 2 + cid
    row_base = wid * ROWS_W

    col_starts = [16 * j for j in range(L // LANES)] + [L - LANES]

    def chunk_body(ci, carry):
        b0 = row_base + ci * RC
        pltpu.sync_copy(x_hbm.at[pl.ds(b0, RC)], idx_v)
        pltpu.sync_copy(sg_hbm.at[pl.ds(b0, RC)], sidx_v)
        lane = lax.iota(jnp.int32, LANES)
        for r in range(RC):
            for c0 in col_starts:
                v = idx_v[r, pl.ds(c0, LANES)]
                q = lax.rem(v, XB)
                tidx_v[r, pl.ds(c0, LANES)] = (
                    v - q + ((q & 255) << 1) + (q >> 8))
        gathers = []
        for r in range(RC):
            for c0, w in HALVES:
                gathers.append(pltpu.async_copy(
                    table_hbm.at[tidx_v.at[r, pl.ds(c0, w)]],
                    rows_v.at[r, pl.ds(c0, w)], sem))
        for r in range(RC):
            for c0 in col_starts:
                seg = sidx_v[r, pl.ds(c0, LANES)]
                cidx_v[r, pl.ds(c0, LANES)] = seg * L + (c0 + lane)
        for cp in gathers:
            cp.wait()
        adds = []
        for r in range(RC):
            for c0, w in HALVES:
                adds.append(pltpu.async_copy(
                    comb_hbm.at[cidx_v.at[r, pl.ds(c0, w)]],
                    rows_v.at[r, pl.ds(c0, w)], sem2, add=True))
        for cp in adds:
            cp.wait()
        pltpu.sync_copy(rows_v, out_hbm.at[pl.ds(b0, RC)])
        return carry

    lax.fori_loop(0, NCH, chunk_body, 0)


def kernel(x, segment_info, table, seg_table, pos):
    xi = x.astype(jnp.int32)
    si = segment_info.astype(jnp.int32)
    comb = _build_comb(seg_table.astype(jnp.float32),
                       pos[:L].astype(jnp.float32)).reshape(3 * L, D)
    t2 = _relayout(table.T)
    t64 = t2.reshape(2 * V2, D)
    return _sc_embed(xi, si, t64, comb)


# speedup vs baseline: 2.8094x; 2.8094x over previous
import functools

import jax
import jax.numpy as jnp
from jax import lax
from jax.experimental import pallas as pl
from jax.experimental.pallas import tpu as pltpu
from jax.experimental.pallas import tpu_sc as plsc

B, L, V, D = 1024, 200, 1000000, 64
NW = 32
ROWS_W = B // NW
RC = 4
NCH = ROWS_W // RC
HALVES = ((0, 104), (104, 96))
LANES = 16
XB = 8192
NXB = (V + XB - 1) // XB
V2 = (XB // 2) * NXB


def _comb_body(seg_ref, pos_ref, out_ref):
    out_ref[...] = seg_ref[...][:, None, :] + pos_ref[...][None, :, :]


def _build_comb(seg_table, pos200):
    return pl.pallas_call(
        _comb_body,
        out_shape=jax.ShapeDtypeStruct((3, L, D), jnp.float32),
    )(seg_table, pos200)


def _relayout_body(tt_ref, out_ref):
    a = jnp.swapaxes(tt_ref[...], 0, 1)
    out_ref[:, :D] = a[: XB // 2, :]
    out_ref[:, D:] = a[XB // 2:, :]


def _relayout(tt):
    return pl.pallas_call(
        _relayout_body,
        grid=(NXB,),
        in_specs=[pl.BlockSpec((D, XB), lambda i: (0, i))],
        out_specs=pl.BlockSpec((XB // 2, 2 * D), lambda i: (i, 0)),
        out_shape=jax.ShapeDtypeStruct((V2, 2 * D), jnp.float32),
    )(tt)


_MESH = plsc.VectorSubcoreMesh(core_axis_name="c", subcore_axis_name="s")


@functools.partial(
    pl.kernel,
    mesh=_MESH,
    compiler_params=pltpu.CompilerParams(use_tc_tiling_on_sc=False),
    out_type=jax.ShapeDtypeStruct((B, L, D), jnp.float32),
    scratch_types=[
        pltpu.VMEM((RC, L), jnp.int32),
        pltpu.VMEM((RC, L), jnp.int32),
        pltpu.VMEM((RC, L), jnp.int32),
        pltpu.VMEM((RC, L), jnp.int32),
        pltpu.VMEM((RC, L, D), jnp.float32),
        pltpu.SemaphoreType.DMA,
        pltpu.SemaphoreType.DMA,
    ],
)
def _sc_embed(x_hbm, sg_hbm, table_hbm, comb_hbm, out_hbm,
              idx_v, tidx_v, sidx_v, cidx_v, rows_v, sem, sem2):
    cid = lax.axis_index("c")
    sid = lax.axis_index("s")
    wid = sid * 2 + cid
    row_base = wid * ROWS_W

    col_starts = [16 * j for j in range(L // LANES)] + [L - LANES]

    def chunk_body(ci, carry):
        b0 = row_base + ci * RC
        pltpu.sync_copy(x_hbm.at[pl.ds(b0, RC)], idx_v)
        pltpu.sync_copy(sg_hbm.at[pl.ds(b0, RC)], sidx_v)
        lane = lax.iota(jnp.int32, LANES)
        half_mask = XB // 2 - 1
        half_shift = (XB // 2).bit_length() - 1
        for r in range(RC):
            for c0 in col_starts:
                v = idx_v[r, pl.ds(c0, LANES)]
                q = v & (XB - 1)
                tidx_v[r, pl.ds(c0, LANES)] = (
                    v - q + ((q & half_mask) << 1) + (q >> half_shift))
        gathers = []
        for r in range(RC):
            for c0, w in HALVES:
                gathers.append(pltpu.async_copy(
                    table_hbm.at[tidx_v.at[r, pl.ds(c0, w)]],
                    rows_v.at[r, pl.ds(c0, w)], sem))
        for r in range(RC):
            for c0 in col_starts:
                seg = sidx_v[r, pl.ds(c0, LANES)]
                cidx_v[r, pl.ds(c0, LANES)] = seg * L + (c0 + lane)
        for cp in gathers:
            cp.wait()
        adds = []
        for r in range(RC):
            for c0, w in HALVES:
                adds.append(pltpu.async_copy(
                    comb_hbm.at[cidx_v.at[r, pl.ds(c0, w)]],
                    rows_v.at[r, pl.ds(c0, w)], sem2, add=True))
        for cp in adds:
            cp.wait()
        pltpu.sync_copy(rows_v, out_hbm.at[pl.ds(b0, RC)])
        return carry

    lax.fori_loop(0, NCH, chunk_body, 0)


def kernel(x, segment_info, table, seg_table, pos):
    xi = x.astype(jnp.int32)
    si = segment_info.astype(jnp.int32)
    comb = _build_comb(seg_table.astype(jnp.float32),
                       pos[:L].astype(jnp.float32)).reshape(3 * L, D)
    t2 = _relayout(table.T)
    t64 = t2.reshape(2 * V2, D)
    return _sc_embed(xi, si, t64, comb)


# speedup vs baseline: 2.8704x; 1.0217x over previous
import functools

import jax
import jax.numpy as jnp
from jax import lax
from jax.experimental import pallas as pl
from jax.experimental.pallas import tpu as pltpu
from jax.experimental.pallas import tpu_sc as plsc

B, L, V, D = 1024, 200, 1000000, 64
NW = 32
ROWS_W = B // NW
RC = 8
NCH = ROWS_W // RC
HALVES = ((0, 104), (104, 96))
LANES = 16
XB = 8192
NXB = (V + XB - 1) // XB
V2 = (XB // 2) * NXB


def _comb_body(seg_ref, pos_ref, out_ref):
    out_ref[...] = seg_ref[...][:, None, :] + pos_ref[...][None, :, :]


def _build_comb(seg_table, pos200):
    return pl.pallas_call(
        _comb_body,
        out_shape=jax.ShapeDtypeStruct((3, L, D), jnp.float32),
    )(seg_table, pos200)


def _relayout_body(tt_ref, out_ref):
    a = jnp.swapaxes(tt_ref[...], 0, 1)
    out_ref[:, :D] = a[: XB // 2, :]
    out_ref[:, D:] = a[XB // 2:, :]


def _relayout(tt):
    return pl.pallas_call(
        _relayout_body,
        grid=(NXB,),
        in_specs=[pl.BlockSpec((D, XB), lambda i: (0, i))],
        out_specs=pl.BlockSpec((XB // 2, 2 * D), lambda i: (i, 0)),
        out_shape=jax.ShapeDtypeStruct((V2, 2 * D), jnp.float32),
    )(tt)


_MESH = plsc.VectorSubcoreMesh(core_axis_name="c", subcore_axis_name="s")


@functools.partial(
    pl.kernel,
    mesh=_MESH,
    compiler_params=pltpu.CompilerParams(use_tc_tiling_on_sc=False),
    out_type=jax.ShapeDtypeStruct((B, L, D), jnp.float32),
    scratch_types=[
        pltpu.VMEM((RC, L), jnp.int32),
        pltpu.VMEM((RC, L), jnp.int32),
        pltpu.VMEM((RC, L), jnp.int32),
        pltpu.VMEM((RC, L), jnp.int32),
        pltpu.VMEM((RC, L, D), jnp.float32),
        pltpu.SemaphoreType.DMA,
        pltpu.SemaphoreType.DMA,
    ],
)
def _sc_embed(x_hbm, sg_hbm, table_hbm, comb_hbm, out_hbm,
              idx_v, tidx_v, sidx_v, cidx_v, rows_v, sem, sem2):
    cid = lax.axis_index("c")
    sid = lax.axis_index("s")
    wid = sid * 2 + cid
    row_base = wid * ROWS_W

    col_starts = [16 * j for j in range(L // LANES)] + [L - LANES]

    def chunk_body(ci, carry):
        b0 = row_base + ci * RC
        pltpu.sync_copy(x_hbm.at[pl.ds(b0, RC)], idx_v)
        pltpu.sync_copy(sg_hbm.at[pl.ds(b0, RC)], sidx_v)
        lane = lax.iota(jnp.int32, LANES)
        half_mask = XB // 2 - 1
        half_shift = (XB // 2).bit_length() - 1
        for r in range(RC):
            for c0 in col_starts:
                v = idx_v[r, pl.ds(c0, LANES)]
                q = v & (XB - 1)
                tidx_v[r, pl.ds(c0, LANES)] = (
                    v - q + ((q & half_mask) << 1) + (q >> half_shift))
        gathers = []
        for r in range(RC):
            for c0, w in HALVES:
                gathers.append(pltpu.async_copy(
                    table_hbm.at[tidx_v.at[r, pl.ds(c0, w)]],
                    rows_v.at[r, pl.ds(c0, w)], sem))
        for r in range(RC):
            for c0 in col_starts:
                seg = sidx_v[r, pl.ds(c0, LANES)]
                cidx_v[r, pl.ds(c0, LANES)] = seg * L + (c0 + lane)
        for cp in gathers:
            cp.wait()
        adds = []
        for r in range(RC):
            for c0, w in HALVES:
                adds.append(pltpu.async_copy(
                    comb_hbm.at[cidx_v.at[r, pl.ds(c0, w)]],
                    rows_v.at[r, pl.ds(c0, w)], sem2, add=True))
        for cp in adds:
            cp.wait()
        pltpu.sync_copy(rows_v, out_hbm.at[pl.ds(b0, RC)])
        return carry

    lax.fori_loop(0, NCH, chunk_body, 0)


def kernel(x, segment_info, table, seg_table, pos):
    xi = x.astype(jnp.int32)
    si = segment_info.astype(jnp.int32)
    comb = _build_comb(seg_table.astype(jnp.float32),
                       pos[:L].astype(jnp.float32)).reshape(3 * L, D)
    t2 = _relayout(table.T)
    t64 = t2.reshape(2 * V2, D)
    return _sc_embed(xi, si, t64, comb)


# speedup vs baseline: 3.0505x; 1.0627x over previous
import functools

import jax
import jax.numpy as jnp
from jax import lax
from jax.experimental import pallas as pl
from jax.experimental.pallas import tpu as pltpu
from jax.experimental.pallas import tpu_sc as plsc

B, L, V, D = 1024, 200, 1000000, 64
NW = 32
ROWS_W = B // NW
RC = 8
NCH = ROWS_W // RC
HALVES = ((0, 104), (104, 96))
LANES = 16
XB = 16384
NXB = (V + XB - 1) // XB
V2 = (XB // 2) * NXB


def _comb_body(seg_ref, pos_ref, out_ref):
    out_ref[...] = seg_ref[...][:, None, :] + pos_ref[...][None, :, :]


def _build_comb(seg_table, pos200):
    return pl.pallas_call(
        _comb_body,
        out_shape=jax.ShapeDtypeStruct((3, L, D), jnp.float32),
    )(seg_table, pos200)


def _relayout_body(tt_ref, out_ref):
    a = jnp.swapaxes(tt_ref[...], 0, 1)
    out_ref[:, :D] = a[: XB // 2, :]
    out_ref[:, D:] = a[XB // 2:, :]


def _relayout(tt):
    return pl.pallas_call(
        _relayout_body,
        grid=(NXB,),
        in_specs=[pl.BlockSpec((D, XB), lambda i: (0, i))],
        out_specs=pl.BlockSpec((XB // 2, 2 * D), lambda i: (i, 0)),
        out_shape=jax.ShapeDtypeStruct((V2, 2 * D), jnp.float32),
    )(tt)


_MESH = plsc.VectorSubcoreMesh(core_axis_name="c", subcore_axis_name="s")


@functools.partial(
    pl.kernel,
    mesh=_MESH,
    compiler_params=pltpu.CompilerParams(use_tc_tiling_on_sc=False),
    out_type=jax.ShapeDtypeStruct((B, L, D), jnp.float32),
    scratch_types=[
        pltpu.VMEM((RC, L), jnp.int32),
        pltpu.VMEM((RC, L), jnp.int32),
        pltpu.VMEM((RC, L), jnp.int32),
        pltpu.VMEM((RC, L), jnp.int32),
        pltpu.VMEM((RC, L, D), jnp.float32),
        pltpu.SemaphoreType.DMA,
        pltpu.SemaphoreType.DMA,
    ],
)
def _sc_embed(x_hbm, sg_hbm, table_hbm, comb_hbm, out_hbm,
              idx_v, tidx_v, sidx_v, cidx_v, rows_v, sem, sem2):
    cid = lax.axis_index("c")
    sid = lax.axis_index("s")
    wid = sid * 2 + cid
    row_base = wid * ROWS_W

    col_starts = [16 * j for j in range(L // LANES)] + [L - LANES]

    def chunk_body(ci, carry):
        b0 = row_base + ci * RC
        pltpu.sync_copy(x_hbm.at[pl.ds(b0, RC)], idx_v)
        pltpu.sync_copy(sg_hbm.at[pl.ds(b0, RC)], sidx_v)
        lane = lax.iota(jnp.int32, LANES)
        half_mask = XB // 2 - 1
        half_shift = (XB // 2).bit_length() - 1
        for r in range(RC):
            for c0 in col_starts:
                v = idx_v[r, pl.ds(c0, LANES)]
                q = v & (XB - 1)
                tidx_v[r, pl.ds(c0, LANES)] = (
                    v - q + ((q & half_mask) << 1) + (q >> half_shift))
        gathers = []
        for r in range(RC):
            for c0, w in HALVES:
                gathers.append(pltpu.async_copy(
                    table_hbm.at[tidx_v.at[r, pl.ds(c0, w)]],
                    rows_v.at[r, pl.ds(c0, w)], sem))
        for r in range(RC):
            for c0 in col_starts:
                seg = sidx_v[r, pl.ds(c0, LANES)]
                cidx_v[r, pl.ds(c0, LANES)] = seg * L + (c0 + lane)
        for cp in gathers:
            cp.wait()
        adds = []
        for r in range(RC):
            for c0, w in HALVES:
                adds.append(pltpu.async_copy(
                    comb_hbm.at[cidx_v.at[r, pl.ds(c0, w)]],
                    rows_v.at[r, pl.ds(c0, w)], sem2, add=True))
        for cp in adds:
            cp.wait()
        pltpu.sync_copy(rows_v, out_hbm.at[pl.ds(b0, RC)])
        return carry

    lax.fori_loop(0, NCH, chunk_body, 0)


def kernel(x, segment_info, table, seg_table, pos):
    xi = x.astype(jnp.int32)
    si = segment_info.astype(jnp.int32)
    comb = _build_comb(seg_table.astype(jnp.float32),
                       pos[:L].astype(jnp.float32)).reshape(3 * L, D)
    t2 = _relayout(table.T)
    t64 = t2.reshape(2 * V2, D)
    return _sc_embed(xi, si, t64, comb)


# speedup vs baseline: 3.1517x; 1.0332x over previous
import functools

import jax
import jax.numpy as jnp
from jax import lax
from jax.experimental import pallas as pl
from jax.experimental.pallas import tpu as pltpu
from jax.experimental.pallas import tpu_sc as plsc

B, L, V, D = 1024, 200, 1000000, 64
NW = 32
ROWS_W = B // NW
RC = 8
NCH = ROWS_W // RC
HALVES = ((0, 104), (104, 96))
LANES = 16
XB = 32768
NXB = (V + XB - 1) // XB
V2 = (XB // 2) * NXB


def _comb_body(seg_ref, pos_ref, out_ref):
    out_ref[...] = seg_ref[...][:, None, :] + pos_ref[...][None, :, :]


def _build_comb(seg_table, pos200):
    return pl.pallas_call(
        _comb_body,
        out_shape=jax.ShapeDtypeStruct((3, L, D), jnp.float32),
    )(seg_table, pos200)


def _relayout_body(tt_ref, out_ref):
    a = jnp.swapaxes(tt_ref[...], 0, 1)
    out_ref[:, :D] = a[: XB // 2, :]
    out_ref[:, D:] = a[XB // 2:, :]


def _relayout(tt):
    return pl.pallas_call(
        _relayout_body,
        grid=(NXB,),
        in_specs=[pl.BlockSpec((D, XB), lambda i: (0, i))],
        out_specs=pl.BlockSpec((XB // 2, 2 * D), lambda i: (i, 0)),
        out_shape=jax.ShapeDtypeStruct((V2, 2 * D), jnp.float32),
    )(tt)


_MESH = plsc.VectorSubcoreMesh(core_axis_name="c", subcore_axis_name="s")


@functools.partial(
    pl.kernel,
    mesh=_MESH,
    compiler_params=pltpu.CompilerParams(use_tc_tiling_on_sc=False),
    out_type=jax.ShapeDtypeStruct((B, L, D), jnp.float32),
    scratch_types=[
        pltpu.VMEM((RC, L), jnp.int32),
        pltpu.VMEM((RC, L), jnp.int32),
        pltpu.VMEM((RC, L), jnp.int32),
        pltpu.VMEM((RC, L), jnp.int32),
        pltpu.VMEM((RC, L, D), jnp.float32),
        pltpu.SemaphoreType.DMA,
        pltpu.SemaphoreType.DMA,
    ],
)
def _sc_embed(x_hbm, sg_hbm, table_hbm, comb_hbm, out_hbm,
              idx_v, tidx_v, sidx_v, cidx_v, rows_v, sem, sem2):
    cid = lax.axis_index("c")
    sid = lax.axis_index("s")
    wid = sid * 2 + cid
    row_base = wid * ROWS_W

    col_starts = [16 * j for j in range(L // LANES)] + [L - LANES]

    def chunk_body(ci, carry):
        b0 = row_base + ci * RC
        pltpu.sync_copy(x_hbm.at[pl.ds(b0, RC)], idx_v)
        pltpu.sync_copy(sg_hbm.at[pl.ds(b0, RC)], sidx_v)
        lane = lax.iota(jnp.int32, LANES)
        half_mask = XB // 2 - 1
        half_shift = (XB // 2).bit_length() - 1
        for r in range(RC):
            for c0 in col_starts:
                v = idx_v[r, pl.ds(c0, LANES)]
                q = v & (XB - 1)
                tidx_v[r, pl.ds(c0, LANES)] = (
                    v - q + ((q & half_mask) << 1) + (q >> half_shift))
        gathers = []
        for r in range(RC):
            for c0, w in HALVES:
                gathers.append(pltpu.async_copy(
                    table_hbm.at[tidx_v.at[r, pl.ds(c0, w)]],
                    rows_v.at[r, pl.ds(c0, w)], sem))
        for r in range(RC):
            for c0 in col_starts:
                seg = sidx_v[r, pl.ds(c0, LANES)]
                cidx_v[r, pl.ds(c0, LANES)] = seg * L + (c0 + lane)
        for cp in gathers:
            cp.wait()
        adds = []
        for r in range(RC):
            for c0, w in HALVES:
                adds.append(pltpu.async_copy(
                    comb_hbm.at[cidx_v.at[r, pl.ds(c0, w)]],
                    rows_v.at[r, pl.ds(c0, w)], sem2, add=True))
        for cp in adds:
            cp.wait()
        pltpu.sync_copy(rows_v, out_hbm.at[pl.ds(b0, RC)])
        return carry

    lax.fori_loop(0, NCH, chunk_body, 0)


def kernel(x, segment_info, table, seg_table, pos):
    xi = x.astype(jnp.int32)
    si = segment_info.astype(jnp.int32)
    comb = _build_comb(seg_table.astype(jnp.float32),
                       pos[:L].astype(jnp.float32)).reshape(3 * L, D)
    t2 = _relayout(table.T)
    t64 = t2.reshape(2 * V2, D)
    return _sc_embed(xi, si, t64, comb)
